# Initial kernel scaffold; baseline (speedup 1.0000x reference)
import jax
import jax.numpy as jnp
from jax.experimental import pallas as pl


def kernel(x, Wg, bg, W1, b1, W2, b2):
    raise NotImplementedError("write your pallas kernel here")


# speedup vs baseline: 1.2748x; 1.2748x over previous
import functools

import jax
import jax.numpy as jnp
from jax import lax
from jax.experimental import pallas as pl
from jax.experimental.pallas import tpu as pltpu
from jax.experimental.pallas import tpu_sc as plsc

_RB = 1024
_NW = 32


def _keys_from_scores(u32):
    sign = u32 >> jnp.uint32(31)
    return u32 ^ (jnp.uint32(0x80000000) | (sign * jnp.uint32(0xFFFFFFFF)))


def _gate_body(x_ref, wg_ref, bg_ref, out_ref):
    s = jnp.dot(x_ref[...], wg_ref[...], preferred_element_type=jnp.float32)
    s = s[:, 0] + bg_ref[0]
    out_ref[...] = s.reshape(out_ref.shape)


def _select_body(k, n, s_ref, mask_ref, pi_ref, cnt_ref):
    nrow, ncol = s_ref.shape
    nb = n // _RB
    u = lax.bitcast_convert_type(s_ref[...], jnp.uint32)
    keys = _keys_from_scores(u)
    gidx = (lax.broadcasted_iota(jnp.int32, (nrow, ncol), 0) * ncol
            + lax.broadcasted_iota(jnp.int32, (nrow, ncol), 1))

    def bis(i, t):
        cand = t | lax.shift_left(jnp.uint32(1), (31 - i).astype(jnp.uint32))
        cnt = jnp.sum((keys >= cand).astype(jnp.int32))
        return jnp.where(cnt >= k, cand, t)

    t = lax.fori_loop(0, 32, bis, jnp.uint32(0))
    n_gt = jnp.sum((keys > t).astype(jnp.int32))
    r = k - n_gt
    tie = keys == t

    def bis2(i, lohi):
        lo, hi = lohi
        mid = (lo + hi) // 2
        cnt = jnp.sum((tie & (gidx <= mid)).astype(jnp.int32))
        ge = cnt >= r
        return jnp.where(ge, lo, mid + 1), jnp.where(ge, mid, hi)

    c, _ = lax.fori_loop(0, 17, bis2, (jnp.int32(0), jnp.int32(n - 1)))

    maskb = (keys > t) | (tie & (gidx <= c))
    mask_ref[...] = maskb.astype(jnp.float32)
    m3 = maskb.reshape(nb, _RB // ncol, ncol).astype(jnp.int32)
    cnt_ref[...] = jnp.sum(m3, axis=1)

    def offs(b, acc):
        cb = jnp.sum(cnt_ref[pl.ds(b, 1), :])
        pi_ref[b] = acc
        pi_ref[256 + b] = cb
        return acc + ((cb + 7) // 8) * 8

    total = lax.fori_loop(0, nb, offs, jnp.int32(0))
    pi_ref[128] = total
    pi_ref[129] = c
    pi_ref[130] = lax.bitcast_convert_type(t, jnp.int32)


def _sc_splat(ref, j):
    return plsc.load_gather(ref, [jnp.full((16,), j, jnp.int32)])


def _compact_body(n, k_pad, s_hbm, pi_hbm, idx_hbm, s_v, loc_v, pi_v):
    chunk = n // _NW
    blocks_per_w = chunk // _RB
    wid = lax.axis_index("s") * 2 + lax.axis_index("c")
    pltpu.sync_copy(pi_hbm, pi_v)
    t_splat = plsc.bitcast(_sc_splat(pi_v, 130), jnp.uint32)
    c_splat = _sc_splat(pi_v, 129)
    pltpu.sync_copy(s_hbm.at[pl.ds(pl.multiple_of(wid * chunk, 8), chunk)], s_v)
    iota16 = lax.iota(jnp.int32, 16)
    zeros16 = jnp.zeros((16,), jnp.int32)

    for bl in range(blocks_per_w):
        blk = wid * blocks_per_w + bl
        oo_b = jnp.max(_sc_splat(pi_v, blk))
        base = wid * chunk + bl * _RB

        def step(i, off):
            s = s_v[pl.ds(bl * _RB + i * 16, 16)]
            key = _keys_from_scores(plsc.bitcast(s, jnp.uint32))
            g = iota16 + (base + i * 16)
            m = (key > t_splat) | ((key == t_splat) & (g <= c_splat))
            mi = m.astype(jnp.int32)
            pos = plsc.cumsum(mi) - 1 + off
            plsc.store_scatter(loc_v, [pos], g, mask=m)
            return off + jnp.sum(mi)

        cb = lax.fori_loop(0, _RB // 16, step, jnp.int32(0))
        loc_v[pl.ds(cb, 16)] = zeros16
        n8 = (cb + 7) // 8

        @pl.loop(0, n8)
        def _(j):
            pltpu.sync_copy(loc_v.at[pl.ds(j * 8, 8)],
                            idx_hbm.at[pl.ds(pl.multiple_of(oo_b + j * 8, 8), 8)])

    @pl.when(wid == _NW - 1)
    def _():
        total = jnp.max(_sc_splat(pi_v, 128))
        loc_v[pl.ds(0, 16)] = zeros16
        n_tail = (k_pad - total) // 8

        @pl.loop(0, n_tail)
        def _(j):
            pltpu.sync_copy(loc_v.at[pl.ds(0, 8)],
                            idx_hbm.at[pl.ds(pl.multiple_of(total + j * 8, 8), 8)])


def _gather_body(rpw, x_hbm, idx_hbm, act_hbm, idx_v, rows_v, sem):
    wid = lax.axis_index("s") * 2 + lax.axis_index("c")
    base = pl.multiple_of(wid * rpw, 8)
    pltpu.sync_copy(idx_hbm.at[pl.ds(base, rpw)], idx_v)
    pltpu.async_copy(x_hbm.at[idx_v], rows_v, sem).wait()
    pltpu.sync_copy(rows_v, act_hbm.at[pl.ds(base, rpw)])


def _sc_mesh():
    return plsc.VectorSubcoreMesh(core_axis_name="c", subcore_axis_name="s",
                                  num_cores=2, num_subcores=16)


def _sc_compact(scores_flat, pi, n, k_pad):
    return pl.kernel(
        functools.partial(_compact_body, n, k_pad),
        out_type=jax.ShapeDtypeStruct((k_pad,), jnp.int32),
        mesh=_sc_mesh(),
        scratch_types=[
            pltpu.VMEM((n // _NW,), jnp.float32),
            pltpu.VMEM((_RB + 16,), jnp.int32),
            pltpu.VMEM((512,), jnp.int32),
        ],
        compiler_params=pltpu.CompilerParams(needs_layout_passes=False),
    )(scores_flat, pi)


def _sc_gather(x, idx, k_pad, dim):
    rpw = k_pad // _NW
    return pl.kernel(
        functools.partial(_gather_body, rpw),
        out_type=jax.ShapeDtypeStruct((k_pad, dim), jnp.float32),
        mesh=_sc_mesh(),
        scratch_types=[
            pltpu.VMEM((rpw,), jnp.int32),
            pltpu.VMEM((rpw, dim), jnp.float32),
            pltpu.SemaphoreType.DMA,
        ],
    )(x, idx)


def _mlp_body(a_ref, w1_ref, b1_ref, w2_ref, b2_ref, o_ref):
    h = jnp.dot(a_ref[...], w1_ref[...], preferred_element_type=jnp.float32)
    h = jnp.maximum(h + b1_ref[...][None, :], 0.0)
    o = jnp.dot(h, w2_ref[...], preferred_element_type=jnp.float32)
    o_ref[...] = o + b2_ref[...][None, :]


def _place_body(idx_ref, pi_ref, act_ref, out_ref):
    b = pl.program_id(0)
    out_ref[...] = jnp.zeros(out_ref.shape, jnp.float32)
    lo = pi_ref[b]
    cb = pi_ref[256 + b]

    def body(j, carry):
        p = lo + j
        r = idx_ref[p]
        out_ref[pl.ds(r - b * _RB, 1), :] = act_ref[pl.ds(p, 1), :]
        return carry

    lax.fori_loop(0, cb, body, jnp.int32(0))


def kernel(x, Wg, bg, W1, b1, W2, b2):
    n, dim = x.shape
    k = max(1, int(n * 0.01))
    nb = n // _RB
    k_pad = ((k + nb * 7 + 63) // 64 + 1) * 64
    while (k_pad // _NW) % 8:
        k_pad += 64
    rpw = k_pad // _NW

    scores2d = pl.pallas_call(
        _gate_body,
        grid=(nb,),
        in_specs=[
            pl.BlockSpec((_RB, dim), lambda i: (i, 0)),
            pl.BlockSpec((dim, 1), lambda i: (0, 0)),
            pl.BlockSpec(memory_space=pltpu.SMEM),
        ],
        out_specs=pl.BlockSpec((_RB // 128, 128), lambda i: (i, 0)),
        out_shape=jax.ShapeDtypeStruct((n // 128, 128), jnp.float32),
    )(x, Wg, bg)

    mask2d, pi = pl.pallas_call(
        functools.partial(_select_body, k, n),
        in_specs=[pl.BlockSpec(memory_space=pltpu.VMEM)],
        out_specs=[
            pl.BlockSpec(memory_space=pltpu.VMEM),
            pl.BlockSpec(memory_space=pltpu.SMEM),
        ],
        out_shape=[
            jax.ShapeDtypeStruct((n // 128, 128), jnp.float32),
            jax.ShapeDtypeStruct((512,), jnp.int32),
        ],
        scratch_shapes=[pltpu.VMEM((nb, 128), jnp.int32)],
    )(scores2d)

    idx = _sc_compact(scores2d.reshape(-1), pi, n, k_pad)

    act = _sc_gather(x, idx, k_pad, dim)

    act_out = pl.pallas_call(
        _mlp_body,
        grid=(k_pad // 128,),
        in_specs=[
            pl.BlockSpec((128, dim), lambda i: (i, 0)),
            pl.BlockSpec((dim, dim), lambda i: (0, 0)),
            pl.BlockSpec((dim,), lambda i: (0,)),
            pl.BlockSpec((dim, dim), lambda i: (0, 0)),
            pl.BlockSpec((dim,), lambda i: (0,)),
        ],
        out_specs=pl.BlockSpec((128, dim), lambda i: (i, 0)),
        out_shape=jax.ShapeDtypeStruct((k_pad, dim), jnp.float32),
    )(act, W1, b1, W2, b2)

    out = pl.pallas_call(
        _place_body,
        grid=(nb,),
        in_specs=[
            pl.BlockSpec(memory_space=pltpu.SMEM),
            pl.BlockSpec(memory_space=pltpu.SMEM),
            pl.BlockSpec((k_pad, dim), lambda i: (0, 0)),
        ],
        out_specs=pl.BlockSpec((_RB, dim), lambda i: (i, 0)),
        out_shape=jax.ShapeDtypeStruct((n, dim), jnp.float32),
    )(idx, pi, act_out)

    return out, mask2d.reshape(-1)


# speedup vs baseline: 1.2970x; 1.0174x over previous
import functools

import jax
import jax.numpy as jnp
from jax import lax
from jax.experimental import pallas as pl
from jax.experimental.pallas import tpu as pltpu
from jax.experimental.pallas import tpu_sc as plsc

_RB = 1024
_NW = 32


def _keys_from_scores(u32):
    sign = u32 >> jnp.uint32(31)
    return u32 ^ (jnp.uint32(0x80000000) | (sign * jnp.uint32(0xFFFFFFFF)))


def _gate_select_body(k, n, x_ref, wg_ref, bg_ref, s_out_ref, mask_ref,
                      pi_ref, sacc_ref, cnt_ref):
    b = pl.program_id(0)
    nsteps = pl.num_programs(0)
    s = jnp.dot(x_ref[...], wg_ref[...], preferred_element_type=jnp.float32)
    s2d = (s[:, 0] + bg_ref[0]).reshape(s_out_ref.shape)
    s_out_ref[...] = s2d
    sacc_ref[pl.ds(b * s2d.shape[0], s2d.shape[0]), :] = s2d

    @pl.when(b == nsteps - 1)
    def _():
        _select_tail(k, n, sacc_ref, mask_ref, pi_ref, cnt_ref)


def _select_tail(k, n, s_ref, mask_ref, pi_ref, cnt_ref):
    nrow, ncol = s_ref.shape
    nb = n // _RB
    u = lax.bitcast_convert_type(s_ref[...], jnp.uint32)
    keys = _keys_from_scores(u)
    gidx = (lax.broadcasted_iota(jnp.int32, (nrow, ncol), 0) * ncol
            + lax.broadcasted_iota(jnp.int32, (nrow, ncol), 1))

    def bis(i, t):
        cand = t | lax.shift_left(jnp.uint32(1), (31 - i).astype(jnp.uint32))
        cnt = jnp.sum((keys >= cand).astype(jnp.int32))
        return jnp.where(cnt >= k, cand, t)

    t = lax.fori_loop(0, 32, bis, jnp.uint32(0))
    n_gt = jnp.sum((keys > t).astype(jnp.int32))
    r = k - n_gt
    tie = keys == t

    def bis2(i, lohi):
        lo, hi = lohi
        mid = (lo + hi) // 2
        cnt = jnp.sum((tie & (gidx <= mid)).astype(jnp.int32))
        ge = cnt >= r
        return jnp.where(ge, lo, mid + 1), jnp.where(ge, mid, hi)

    c, _ = lax.fori_loop(0, 17, bis2, (jnp.int32(0), jnp.int32(n - 1)))

    maskb = (keys > t) | (tie & (gidx <= c))
    mask_ref[...] = maskb.astype(jnp.float32)
    m3 = maskb.reshape(nb, _RB // ncol, ncol).astype(jnp.int32)
    cnt_ref[...] = jnp.sum(m3, axis=1)

    def offs(b, acc):
        cb = jnp.sum(cnt_ref[pl.ds(b, 1), :])
        pi_ref[b] = acc
        pi_ref[256 + b] = cb
        return acc + ((cb + 7) // 8) * 8

    total = lax.fori_loop(0, nb, offs, jnp.int32(0))
    pi_ref[128] = total
    pi_ref[129] = c
    pi_ref[130] = lax.bitcast_convert_type(t, jnp.int32)


def _sc_splat(ref, j):
    return plsc.load_gather(ref, [jnp.full((16,), j, jnp.int32)])


def _compact_body(n, k_pad, s_hbm, pi_hbm, idx_hbm, s_v, loc_v, pi_v):
    chunk = n // _NW
    blocks_per_w = chunk // _RB
    wid = lax.axis_index("s") * 2 + lax.axis_index("c")
    pltpu.sync_copy(pi_hbm, pi_v)
    t_splat = plsc.bitcast(_sc_splat(pi_v, 130), jnp.uint32)
    c_splat = _sc_splat(pi_v, 129)
    pltpu.sync_copy(s_hbm.at[pl.ds(pl.multiple_of(wid * chunk, 8), chunk)], s_v)
    iota16 = lax.iota(jnp.int32, 16)
    zeros16 = jnp.zeros((16,), jnp.int32)

    for bl in range(blocks_per_w):
        blk = wid * blocks_per_w + bl
        oo_b = jnp.max(_sc_splat(pi_v, blk))
        base = wid * chunk + bl * _RB

        def step(i, off):
            s = s_v[pl.ds(bl * _RB + i * 16, 16)]
            key = _keys_from_scores(plsc.bitcast(s, jnp.uint32))
            g = iota16 + (base + i * 16)
            m = (key > t_splat) | ((key == t_splat) & (g <= c_splat))
            mi = m.astype(jnp.int32)
            pos = plsc.cumsum(mi) - 1 + off
            plsc.store_scatter(loc_v, [pos], g, mask=m)
            return off + jnp.sum(mi)

        cb = lax.fori_loop(0, _RB // 16, step, jnp.int32(0))
        loc_v[pl.ds(cb, 16)] = zeros16
        n8 = (cb + 7) // 8

        @pl.loop(0, n8)
        def _(j):
            pltpu.sync_copy(loc_v.at[pl.ds(j * 8, 8)],
                            idx_hbm.at[pl.ds(pl.multiple_of(oo_b + j * 8, 8), 8)])

    @pl.when(wid == _NW - 1)
    def _():
        total = jnp.max(_sc_splat(pi_v, 128))
        loc_v[pl.ds(0, 16)] = zeros16
        n_tail = (k_pad - total) // 8

        @pl.loop(0, n_tail)
        def _(j):
            pltpu.sync_copy(loc_v.at[pl.ds(0, 8)],
                            idx_hbm.at[pl.ds(pl.multiple_of(total + j * 8, 8), 8)])


def _gather_body(rpw, x_hbm, idx_hbm, act_hbm, idx_v, rows_v, sem):
    wid = lax.axis_index("s") * 2 + lax.axis_index("c")
    base = pl.multiple_of(wid * rpw, 8)
    pltpu.sync_copy(idx_hbm.at[pl.ds(base, rpw)], idx_v)
    pltpu.async_copy(x_hbm.at[idx_v], rows_v, sem).wait()
    pltpu.sync_copy(rows_v, act_hbm.at[pl.ds(base, rpw)])


def _sc_mesh():
    return plsc.VectorSubcoreMesh(core_axis_name="c", subcore_axis_name="s",
                                  num_cores=2, num_subcores=16)


def _sc_compact(scores_flat, pi, n, k_pad):
    return pl.kernel(
        functools.partial(_compact_body, n, k_pad),
        out_type=jax.ShapeDtypeStruct((k_pad,), jnp.int32),
        mesh=_sc_mesh(),
        scratch_types=[
            pltpu.VMEM((n // _NW,), jnp.float32),
            pltpu.VMEM((_RB + 16,), jnp.int32),
            pltpu.VMEM((512,), jnp.int32),
        ],
        compiler_params=pltpu.CompilerParams(needs_layout_passes=False),
    )(scores_flat, pi)


def _sc_gather(x, idx, k_pad, dim):
    rpw = k_pad // _NW
    return pl.kernel(
        functools.partial(_gather_body, rpw),
        out_type=jax.ShapeDtypeStruct((k_pad, dim), jnp.float32),
        mesh=_sc_mesh(),
        scratch_types=[
            pltpu.VMEM((rpw,), jnp.int32),
            pltpu.VMEM((rpw, dim), jnp.float32),
            pltpu.SemaphoreType.DMA,
        ],
    )(x, idx)


def _dot3(xf32, wh_ref, wl_ref):
    xh = xf32.astype(jnp.bfloat16)
    xl = (xf32 - xh.astype(jnp.float32)).astype(jnp.bfloat16)
    return (jnp.dot(xh, wh_ref[...], preferred_element_type=jnp.float32)
            + jnp.dot(xh, wl_ref[...], preferred_element_type=jnp.float32)
            + jnp.dot(xl, wh_ref[...], preferred_element_type=jnp.float32))


def _mlp_place_body(idx_ref, pi_ref, a_ref, w1h_ref, w1l_ref, w2h_ref,
                    w2l_ref, b1_ref, b2_ref, out_ref, ao_ref):
    b = pl.program_id(0)

    @pl.when(b == 0)
    def _():
        h = _dot3(a_ref[...], w1h_ref, w1l_ref)
        h = jnp.maximum(h + b1_ref[...][None, :], 0.0)
        o = _dot3(h, w2h_ref, w2l_ref)
        ao_ref[...] = o + b2_ref[...][None, :]

    out_ref[...] = jnp.zeros(out_ref.shape, jnp.float32)
    lo = pi_ref[b]
    cb = pi_ref[256 + b]

    def body(j, carry):
        p = lo + j
        r = idx_ref[p]
        out_ref[pl.ds(r - b * _RB, 1), :] = ao_ref[pl.ds(p, 1), :]
        return carry

    lax.fori_loop(0, cb, body, jnp.int32(0))


def kernel(x, Wg, bg, W1, b1, W2, b2):
    n, dim = x.shape
    k = max(1, int(n * 0.01))
    nb = n // _RB
    k_pad = ((k + nb * 7 + 63) // 64 + 1) * 64
    while (k_pad // _NW) % 8:
        k_pad += 64
    rpw = k_pad // _NW

    scores2d, mask2d, pi = pl.pallas_call(
        functools.partial(_gate_select_body, k, n),
        grid=(nb,),
        in_specs=[
            pl.BlockSpec((_RB, dim), lambda i: (i, 0)),
            pl.BlockSpec((dim, 1), lambda i: (0, 0)),
            pl.BlockSpec(memory_space=pltpu.SMEM),
        ],
        out_specs=[
            pl.BlockSpec((_RB // 128, 128), lambda i: (i, 0)),
            pl.BlockSpec((n // 128, 128), lambda i: (0, 0)),
            pl.BlockSpec(memory_space=pltpu.SMEM),
        ],
        out_shape=[
            jax.ShapeDtypeStruct((n // 128, 128), jnp.float32),
            jax.ShapeDtypeStruct((n // 128, 128), jnp.float32),
            jax.ShapeDtypeStruct((512,), jnp.int32),
        ],
        scratch_shapes=[
            pltpu.VMEM((n // 128, 128), jnp.float32),
            pltpu.VMEM((nb, 128), jnp.int32),
        ],
    )(x, Wg, bg)

    idx = _sc_compact(scores2d.reshape(-1), pi, n, k_pad)

    act = _sc_gather(x, idx, k_pad, dim)

    w1h = W1.astype(jnp.bfloat16)
    w1l = (W1 - w1h.astype(jnp.float32)).astype(jnp.bfloat16)
    w2h = W2.astype(jnp.bfloat16)
    w2l = (W2 - w2h.astype(jnp.float32)).astype(jnp.bfloat16)

    out = pl.pallas_call(
        _mlp_place_body,
        grid=(nb,),
        in_specs=[
            pl.BlockSpec(memory_space=pltpu.SMEM),
            pl.BlockSpec(memory_space=pltpu.SMEM),
            pl.BlockSpec((k_pad, dim), lambda i: (0, 0)),
            pl.BlockSpec((dim, dim), lambda i: (0, 0)),
            pl.BlockSpec((dim, dim), lambda i: (0, 0)),
            pl.BlockSpec((dim, dim), lambda i: (0, 0)),
            pl.BlockSpec((dim, dim), lambda i: (0, 0)),
            pl.BlockSpec((dim,), lambda i: (0,)),
            pl.BlockSpec((dim,), lambda i: (0,)),
        ],
        out_specs=pl.BlockSpec((_RB, dim), lambda i: (i, 0)),
        out_shape=jax.ShapeDtypeStruct((n, dim), jnp.float32),
        scratch_shapes=[pltpu.VMEM((k_pad, dim), jnp.float32)],
    )(idx, pi, act, w1h, w1l, w2h, w2l, b1, b2)

    return out, mask2d.reshape(-1)


# speedup vs baseline: 1.7931x; 1.3825x over previous
import functools

import jax
import jax.numpy as jnp
from jax import lax
from jax.experimental import pallas as pl
from jax.experimental.pallas import tpu as pltpu
from jax.experimental.pallas import tpu_sc as plsc

_RB = 1024
_NW = 32


def _keys_from_scores(u32):
    sign = u32 >> jnp.uint32(31)
    return u32 ^ (jnp.uint32(0x80000000) | (sign * jnp.uint32(0xFFFFFFFF)))


def _gate_body(x_ref, wg_ref, bg_ref, out_ref):
    s = jnp.dot(x_ref[...], wg_ref[...], preferred_element_type=jnp.float32)
    s = s[:, 0] + bg_ref[0]
    out_ref[...] = s.reshape(out_ref.shape)


def _select_body(k, n, s_ref, mask_ref, pi_ref, cnt_ref):
    nrow, ncol = s_ref.shape
    nb = n // _RB
    u = lax.bitcast_convert_type(s_ref[...], jnp.uint32)
    keys = _keys_from_scores(u)
    gidx = (lax.broadcasted_iota(jnp.int32, (nrow, ncol), 0) * ncol
            + lax.broadcasted_iota(jnp.int32, (nrow, ncol), 1))

    def bis(i, t):
        cand = t | lax.shift_left(jnp.uint32(1), (31 - i).astype(jnp.uint32))
        cnt = jnp.sum((keys >= cand).astype(jnp.int32))
        return jnp.where(cnt >= k, cand, t)

    t = lax.fori_loop(0, 32, bis, jnp.uint32(0))
    n_gt = jnp.sum((keys > t).astype(jnp.int32))
    r = k - n_gt
    tie = keys == t

    def bis2(i, lohi):
        lo, hi = lohi
        mid = (lo + hi) // 2
        cnt = jnp.sum((tie & (gidx <= mid)).astype(jnp.int32))
        ge = cnt >= r
        return jnp.where(ge, lo, mid + 1), jnp.where(ge, mid, hi)

    n_ge = jnp.sum((keys >= t).astype(jnp.int32))

    def tie_cut():
        c, _ = lax.fori_loop(0, 17, bis2, (jnp.int32(0), jnp.int32(n - 1)))
        return c

    c = lax.cond(n_ge > k, tie_cut, lambda: jnp.int32(n - 1))

    maskb = (keys > t) | (tie & (gidx <= c))
    mask_ref[...] = maskb.astype(jnp.float32)
    m3 = maskb.reshape(nb, _RB // ncol, ncol).astype(jnp.int32)
    cnt_ref[...] = jnp.sum(m3, axis=1)

    bpw = nb // _NW

    def offs(b, acc):
        cb = jnp.sum(cnt_ref[pl.ds(b, 1), :])
        pi_ref[b] = acc
        pi_ref[256 + b] = cb
        acc = acc + cb
        return jnp.where(b % bpw == bpw - 1, ((acc + 7) // 8) * 8, acc)

    total = lax.fori_loop(0, nb, offs, jnp.int32(0))
    pi_ref[128] = total
    pi_ref[129] = c
    pi_ref[130] = lax.bitcast_convert_type(t, jnp.int32)
    pi_ref[131] = jnp.min(jnp.where(maskb, jnp.int32(n), gidx))


def _sc_splat(ref, j):
    return plsc.load_gather(ref, [jnp.full((16,), j, jnp.int32)])


def _compact_body(n, k_pad, s_hbm, pi_hbm, idx_hbm, val_hbm, s_v, loc_v,
                  lval_v, pi_v):
    chunk = n // _NW
    bpw = chunk // _RB
    wid = lax.axis_index("s") * 2 + lax.axis_index("c")
    pltpu.sync_copy(pi_hbm, pi_v)
    t_splat = plsc.bitcast(_sc_splat(pi_v, 130), jnp.uint32)
    c_splat = _sc_splat(pi_v, 129)
    inact16 = _sc_splat(pi_v, 131)
    pltpu.sync_copy(s_hbm.at[pl.ds(pl.multiple_of(wid * chunk, 8), chunk)], s_v)
    iota16 = lax.iota(jnp.int32, 16)
    zeros16f = jnp.zeros((16,), jnp.float32)
    ones16f = jnp.ones((16,), jnp.float32)
    base = wid * chunk
    o_w = jnp.max(_sc_splat(pi_v, wid * bpw))

    def step(i, off):
        s = s_v[pl.ds(i * 16, 16)]
        key = _keys_from_scores(plsc.bitcast(s, jnp.uint32))
        g = iota16 + (base + i * 16)
        m = (key > t_splat) | ((key == t_splat) & (g <= c_splat))
        mi = m.astype(jnp.int32)
        pos = plsc.cumsum(mi) - 1 + off
        plsc.store_scatter(loc_v, [pos], g, mask=m)
        plsc.store_scatter(lval_v, [pos], ones16f, mask=m)
        return off + jnp.sum(mi)

    cw = lax.fori_loop(0, chunk // 16, step, jnp.int32(0), unroll=4)
    loc_v[pl.ds(cw, 16)] = inact16
    lval_v[pl.ds(cw, 16)] = zeros16f
    n8 = (cw + 7) // 8

    @pl.loop(0, n8)
    def _(j):
        pltpu.sync_copy(loc_v.at[pl.ds(j * 8, 8)],
                        idx_hbm.at[pl.ds(pl.multiple_of(o_w + j * 8, 8), 8)])
        pltpu.sync_copy(lval_v.at[pl.ds(j * 8, 8)],
                        val_hbm.at[pl.ds(pl.multiple_of(o_w + j * 8, 8), 8)])

    @pl.when(wid == _NW - 1)
    def _():
        total = jnp.max(_sc_splat(pi_v, 128))
        loc_v[pl.ds(0, 16)] = inact16
        lval_v[pl.ds(0, 16)] = zeros16f
        n_tail = (k_pad - total) // 8

        @pl.loop(0, n_tail)
        def _(j):
            pltpu.sync_copy(loc_v.at[pl.ds(0, 8)],
                            idx_hbm.at[pl.ds(pl.multiple_of(total + j * 8, 8), 8)])
            pltpu.sync_copy(lval_v.at[pl.ds(0, 8)],
                            val_hbm.at[pl.ds(pl.multiple_of(total + j * 8, 8), 8)])


def _gather_body(rpw, x_hbm, idx_hbm, act_hbm, idx_v, rows_v, sem):
    wid = lax.axis_index("s") * 2 + lax.axis_index("c")
    base = pl.multiple_of(wid * rpw, 8)
    pltpu.sync_copy(idx_hbm.at[pl.ds(base, rpw)], idx_v)
    pltpu.async_copy(x_hbm.at[idx_v], rows_v, sem).wait()
    pltpu.sync_copy(rows_v, act_hbm.at[pl.ds(base, rpw)])


def _sc_mesh():
    return plsc.VectorSubcoreMesh(core_axis_name="c", subcore_axis_name="s",
                                  num_cores=2, num_subcores=16)


def _sc_compact(scores_flat, pi, n, k_pad):
    return pl.kernel(
        functools.partial(_compact_body, n, k_pad),
        out_type=(jax.ShapeDtypeStruct((k_pad,), jnp.int32),
                  jax.ShapeDtypeStruct((k_pad,), jnp.float32)),
        mesh=_sc_mesh(),
        scratch_types=[
            pltpu.VMEM((n // _NW,), jnp.float32),
            pltpu.VMEM((n // _NW + 16,), jnp.int32),
            pltpu.VMEM((n // _NW + 16,), jnp.float32),
            pltpu.VMEM((512,), jnp.int32),
        ],
        compiler_params=pltpu.CompilerParams(needs_layout_passes=False),
    )(scores_flat, pi)


def _scatter_body(rpw, zb_hbm, ao_hbm, idx_hbm, idx_v, rows_v):
    wid = lax.axis_index("s") * 2 + lax.axis_index("c")
    base = pl.multiple_of(wid * rpw, 8)
    pltpu.sync_copy(idx_hbm.at[pl.ds(base, rpw)], idx_v)
    pltpu.sync_copy(ao_hbm.at[pl.ds(base, rpw)], rows_v)
    for j in range(rpw // 16):
        vec = idx_v[pl.ds(j * 16, 16)]
        pltpu.sync_copy(rows_v.at[pl.ds(j * 16, 16)], zb_hbm.at[vec])


def _sc_scatter(zb_ref, ao, idx, k_pad, dim):
    rpw = k_pad // _NW
    pl.kernel(
        functools.partial(_scatter_body, rpw),
        out_type=(),
        mesh=_sc_mesh(),
        scratch_types=[
            pltpu.VMEM((rpw,), jnp.int32),
            pltpu.VMEM((rpw, dim), jnp.float32),
        ],
        compiler_params=pltpu.CompilerParams(needs_layout_passes=False),
    )(zb_ref, ao, idx)


def _sc_gather(x, idx, k_pad, dim):
    rpw = k_pad // _NW
    return pl.kernel(
        functools.partial(_gather_body, rpw),
        out_type=jax.ShapeDtypeStruct((k_pad, dim), jnp.float32),
        mesh=_sc_mesh(),
        scratch_types=[
            pltpu.VMEM((rpw,), jnp.int32),
            pltpu.VMEM((rpw, dim), jnp.float32),
            pltpu.SemaphoreType.DMA,
        ],
    )(x, idx)


def _dot3(xf32, wh_ref, wl_ref):
    xh = xf32.astype(jnp.bfloat16)
    xl = (xf32 - xh.astype(jnp.float32)).astype(jnp.bfloat16)
    return (jnp.dot(xh, wh_ref[...], preferred_element_type=jnp.float32)
            + jnp.dot(xh, wl_ref[...], preferred_element_type=jnp.float32)
            + jnp.dot(xl, wh_ref[...], preferred_element_type=jnp.float32))


def _zero_body(out_ref):
    out_ref[...] = jnp.zeros(out_ref.shape, jnp.float32)


def _mlp_body(a_ref, val_ref, w1h_ref, w1l_ref, w2h_ref, w2l_ref, b1_ref,
              b2_ref, ao_ref):
    h = _dot3(a_ref[...], w1h_ref, w1l_ref)
    h = jnp.maximum(h + b1_ref[...][None, :], 0.0)
    o = _dot3(h, w2h_ref, w2l_ref)
    ao_ref[...] = (o + b2_ref[...][None, :]) * val_ref[...]


def kernel(x, Wg, bg, W1, b1, W2, b2):
    n, dim = x.shape
    k = max(1, int(n * 0.01))
    nb = n // _RB
    k_pad = ((k + _NW * 7) + 255) // 256 * 256

    grows = 8192
    scores2d = pl.pallas_call(
        _gate_body,
        grid=(n // grows,),
        in_specs=[
            pl.BlockSpec((grows, dim), lambda i: (i, 0)),
            pl.BlockSpec((dim, 1), lambda i: (0, 0)),
            pl.BlockSpec(memory_space=pltpu.SMEM),
        ],
        out_specs=pl.BlockSpec((grows // 128, 128), lambda i: (i, 0)),
        out_shape=jax.ShapeDtypeStruct((n // 128, 128), jnp.float32),
    )(x, Wg, bg)

    mask2d, pi = pl.pallas_call(
        functools.partial(_select_body, k, n),
        in_specs=[pl.BlockSpec(memory_space=pltpu.VMEM)],
        out_specs=[
            pl.BlockSpec(memory_space=pltpu.VMEM),
            pl.BlockSpec(memory_space=pltpu.SMEM),
        ],
        out_shape=[
            jax.ShapeDtypeStruct((n // 128, 128), jnp.float32),
            jax.ShapeDtypeStruct((512,), jnp.int32),
        ],
        scratch_shapes=[pltpu.VMEM((nb, 128), jnp.int32)],
    )(scores2d)

    idx, val = _sc_compact(scores2d.reshape(-1), pi, n, k_pad)

    act = _sc_gather(x, idx, k_pad, dim)

    w1h = W1.astype(jnp.bfloat16)
    w1l = (W1 - w1h.astype(jnp.float32)).astype(jnp.bfloat16)
    w2h = W2.astype(jnp.bfloat16)
    w2l = (W2 - w2h.astype(jnp.float32)).astype(jnp.bfloat16)

    zrows = 8192
    zerobuf = pl.pallas_call(
        _zero_body,
        grid=(n // zrows,),
        out_specs=pl.BlockSpec((zrows, dim), lambda i: (i, 0)),
        out_shape=jax.ShapeDtypeStruct((n, dim), jnp.float32),
    )()

    ao = pl.pallas_call(
        _mlp_body,
        out_shape=jax.ShapeDtypeStruct((k_pad, dim), jnp.float32),
    )(act, val.reshape(k_pad, 1), w1h, w1l, w2h, w2l, b1, b2)

    zb_ref = jax.new_ref(zerobuf)
    _sc_scatter(zb_ref, ao, idx, k_pad, dim)
    out = zb_ref[...]

    return out, mask2d.reshape(-1)
